# baseline (speedup 1.0000x reference)
import dataclasses
import functools

import jax
import jax.numpy as jnp
from jax import lax
from jax.experimental import pallas as pl
from jax.experimental.pallas import tpu as pltpu
from jax.experimental.pallas import tpu_sc as plsc

B = 16384
D = 64
L = 16
NC = 2
NS = 16
NW = NC * NS
BPW = B // NW


def _mf_kernel(uid_hbm, iid_hbm, ue_hbm, ie_hbm, ub_hbm, ib_hbm, out_hbm,
               uid_v, iid_v, urows_v, irows_v, ub_v, ib_v, out_v, sem):
    wid = lax.axis_index("s") * NC + lax.axis_index("c")
    base = wid * BPW
    pltpu.sync_copy(uid_hbm.at[pl.ds(base, BPW)], uid_v)
    pltpu.sync_copy(iid_hbm.at[pl.ds(base, BPW)], iid_v)
    cp1 = pltpu.async_copy(ue_hbm.at[uid_v], urows_v, sem)
    cp2 = pltpu.async_copy(ie_hbm.at[iid_v], irows_v, sem)
    cp3 = pltpu.async_copy(ub_hbm.at[uid_v], ub_v, sem)
    cp4 = pltpu.async_copy(ib_hbm.at[iid_v], ib_v, sem)
    cp1.wait()
    cp2.wait()
    cp3.wait()
    cp4.wait()

    iota = lax.iota(jnp.int32, L)

    @pl.loop(0, BPW, step=L)
    def _(rb):
        row_idx = rb + iota
        acc = ub_v[pl.ds(rb, L)] + ib_v[pl.ds(rb, L)]
        for k in range(D):
            ck = jnp.full((L,), k, jnp.int32)
            acc = acc + (plsc.load_gather(urows_v, [row_idx, ck])
                         * plsc.load_gather(irows_v, [row_idx, ck]))
        out_v[pl.ds(rb, L)] = acc

    pltpu.sync_copy(out_v, out_hbm.at[pl.ds(base, BPW)])


@jax.jit
def _mf(user_ids, item_ids, user_emb, item_emb, user_biases, item_biases):
    mesh = plsc.VectorSubcoreMesh(core_axis_name="c", subcore_axis_name="s")
    cp = pltpu.CompilerParams()
    if "needs_layout_passes" in pltpu.CompilerParams.__dataclass_fields__:
        cp = dataclasses.replace(cp, needs_layout_passes=False)
    cp = dataclasses.replace(cp, use_tc_tiling_on_sc=False)
    kfn = pl.kernel(
        _mf_kernel,
        mesh=mesh,
        compiler_params=cp,
        out_type=jax.ShapeDtypeStruct((B,), jnp.float32),
        scratch_types=[
            pltpu.VMEM((BPW,), jnp.int32),
            pltpu.VMEM((BPW,), jnp.int32),
            pltpu.VMEM((BPW, D), jnp.float32),
            pltpu.VMEM((BPW, D), jnp.float32),
            pltpu.VMEM((BPW,), jnp.float32),
            pltpu.VMEM((BPW,), jnp.float32),
            pltpu.VMEM((BPW,), jnp.float32),
            pltpu.SemaphoreType.DMA,
        ],
    )
    return kfn(user_ids, item_ids, user_emb, item_emb,
               user_biases, item_biases)


def kernel(user_ids, item_ids, user_emb, item_emb, user_biases, item_biases):
    return _mf(user_ids.astype(jnp.int32), item_ids.astype(jnp.int32),
               user_emb, item_emb,
               user_biases.reshape(-1), item_biases.reshape(-1))


# speedup vs baseline: 1.0021x; 1.0021x over previous
import dataclasses
import functools

import jax
import jax.numpy as jnp
from jax import lax
from jax.experimental import pallas as pl
from jax.experimental.pallas import tpu as pltpu
from jax.experimental.pallas import tpu_sc as plsc

B = 16384
D = 64
L = 16
NC = 2
NS = 16
NW = NC * NS
BPW = B // NW


def _mf_kernel(uid_hbm, iid_hbm, ue_hbm, ie_hbm, ub_hbm, ib_hbm, out_hbm,
               uid_v, iid_v, urows_v, irows_v, ub_v, ib_v, out_v, sem):
    wid = lax.axis_index("s") * NC + lax.axis_index("c")
    base = wid * BPW
    pltpu.sync_copy(uid_hbm.at[pl.ds(base, BPW)], uid_v)
    pltpu.sync_copy(iid_hbm.at[pl.ds(base, BPW)], iid_v)
    cp1 = pltpu.async_copy(ue_hbm.at[uid_v], urows_v, sem)
    cp2 = pltpu.async_copy(ie_hbm.at[iid_v], irows_v, sem)
    cp3 = pltpu.async_copy(ub_hbm.at[uid_v], ub_v, sem)
    cp4 = pltpu.async_copy(ib_hbm.at[iid_v], ib_v, sem)
    cp1.wait()
    cp2.wait()
    cp3.wait()
    cp4.wait()

    iota = lax.iota(jnp.int32, L)
    zero = jnp.zeros((L,), jnp.int32)

    @pl.loop(0, BPW, step=L)
    def _(rb):
        row_idx = rb + iota
        acc = ub_v[pl.ds(rb, L)] + ib_v[pl.ds(rb, L)]
        for k in range(D):
            ck = jnp.full((L,), k, jnp.int32)
            acc = acc + (plsc.load_gather(urows_v, [row_idx, ck])
                         * plsc.load_gather(irows_v, [row_idx, ck]))
        out_v[pl.ds(rb, L)] = acc

    pltpu.sync_copy(out_v, out_hbm.at[pl.ds(base, BPW)])


@jax.jit
def _mf(user_ids, item_ids, user_emb, item_emb, user_biases, item_biases):
    user_biases = user_biases.reshape(-1)
    item_biases = item_biases.reshape(-1)
    mesh = plsc.VectorSubcoreMesh(core_axis_name="c", subcore_axis_name="s")
    cp = pltpu.CompilerParams()
    if "needs_layout_passes" in pltpu.CompilerParams.__dataclass_fields__:
        cp = dataclasses.replace(cp, needs_layout_passes=False)
    cp = dataclasses.replace(cp, use_tc_tiling_on_sc=False)
    kfn = pl.kernel(
        _mf_kernel,
        mesh=mesh,
        compiler_params=cp,
        out_type=jax.ShapeDtypeStruct((B,), jnp.float32),
        scratch_types=[
            pltpu.VMEM((BPW,), jnp.int32),
            pltpu.VMEM((BPW,), jnp.int32),
            pltpu.VMEM((BPW, D), jnp.float32),
            pltpu.VMEM((BPW, D), jnp.float32),
            pltpu.VMEM((BPW,), jnp.float32),
            pltpu.VMEM((BPW,), jnp.float32),
            pltpu.VMEM((BPW,), jnp.float32),
            pltpu.SemaphoreType.DMA,
        ],
    )
    return kfn(user_ids, item_ids, user_emb, item_emb,
               user_biases, item_biases)


def kernel(user_ids, item_ids, user_emb, item_emb, user_biases, item_biases):
    return _mf(user_ids.astype(jnp.int32), item_ids.astype(jnp.int32),
               user_emb, item_emb, user_biases, item_biases)
